# baseline (speedup 1.0000x reference)
import jax
import jax.numpy as jnp
import numpy as np
from jax import lax
from jax.experimental import pallas as pl
from jax.experimental.pallas import tpu as pltpu
from jax.experimental.pallas import tpu_sc as plsc

N = 16384
W = 200
D = 64
DPK = 32
L = 16
NC, NS = 2, 16
NW = NC * NS
ROWS_PER_W = N // NW
CHUNK = 64
NCHUNK = ROWS_PER_W // CHUNK
NVD = D // L
NSLOT = 8


def _fill_idx(idx_buf, j, idxv):
    for v in range(W // L):
        sl = pl.ds(v * L, L)
        idxv[sl] = idx_buf[j, sl] + 1
    tl = pl.ds(W - L, L)
    idxv[tl] = idx_buf[j, tl] + 1


def _gather_start(table_hbm, idxv, rows_ref, sem):
    pltpu.make_async_copy(table_hbm.at[idxv], rows_ref, sem).start()


def _gather_wait(table_hbm, idxv, rows_ref, sem):
    pltpu.make_async_copy(table_hbm.at[idxv], rows_ref, sem).wait()


def _reduce_row(rows_ref, out_ref, r):
    def body(w, accs):
        a = list(accs)
        for p in range(2):
            for g in range(2):
                x = rows_ref[2 * w + p, pl.ds(g * L, L)]
                xb = plsc.bitcast(x, jnp.bfloat16)
                lo, hi = plsc.unpack(xb, format=plsc.PackFormat.INTERLEAVED,
                                     preferred_element_type=jnp.float32)
                a[4 * p + 2 * g] = a[4 * p + 2 * g] + lo
                a[4 * p + 2 * g + 1] = a[4 * p + 2 * g + 1] + hi
        return tuple(a)

    z = jnp.zeros((L,), jnp.float32)
    accs = lax.fori_loop(0, W // 2, body, (z,) * (2 * NVD), unroll=4)
    scale = jnp.float32(1.0 / W)
    for d in range(NVD):
        out_ref[r, pl.ds(d * L, L)] = (accs[d] + accs[NVD + d]) * scale


def _sc_body(state_hbm, table_hbm, out_hbm, idx_buf, out_buf, *rest):
    rows = rest[:NSLOT]
    idxvs = rest[NSLOT:2 * NSLOT]
    sems = rest[2 * NSLOT:]
    wid = lax.axis_index("s") * NC + lax.axis_index("c")
    base = wid * ROWS_PER_W

    def chunk_body(c, _):
        row0 = base + c * CHUNK
        pltpu.sync_copy(state_hbm.at[pl.ds(row0, CHUNK)], idx_buf)

        for k in range(NSLOT):
            _fill_idx(idx_buf, k, idxvs[k])
            _gather_start(table_hbm, idxvs[k], rows[k], sems[k])

        def ring(i, _):
            for k in range(NSLOT):
                r = NSLOT * i + k
                _gather_wait(table_hbm, idxvs[k], rows[k], sems[k])
                @pl.when(i < CHUNK // NSLOT - 1)
                def _():
                    _fill_idx(idx_buf, r + NSLOT, idxvs[k])
                    _gather_start(table_hbm, idxvs[k], rows[k], sems[k])
                _reduce_row(rows[k], out_buf, r)
            return 0
        lax.fori_loop(0, CHUNK // NSLOT, ring, 0)

        pltpu.sync_copy(out_buf, out_hbm.at[pl.ds(row0, CHUNK)])
        return 0

    lax.fori_loop(0, NCHUNK, chunk_body, 0)


_UNPERM = np.array([32 * (c // 32) + 16 * (c % 2) + (c % 32) // 2
                    for c in range(D)], dtype=np.int32)


def kernel(state, table):
    state2 = state.reshape(N, 2 * W).astype(jnp.int32)
    rows_n = table.shape[0]
    tb = lax.bitcast_convert_type(
        table.astype(jnp.bfloat16).reshape(rows_n, DPK, 2), jnp.float32)
    f = pl.kernel(
        _sc_body,
        out_type=jax.ShapeDtypeStruct((N, D), jnp.float32),
        mesh=plsc.VectorSubcoreMesh(core_axis_name="c", subcore_axis_name="s"),
        scratch_types=[
            pltpu.VMEM((CHUNK, 2 * W), jnp.int32),
            pltpu.VMEM((CHUNK, D), jnp.float32),
        ] + [pltpu.VMEM((W, DPK), jnp.float32)] * NSLOT
          + [pltpu.VMEM((W,), jnp.int32)] * NSLOT
          + [pltpu.SemaphoreType.DMA] * NSLOT,
        compiler_params=pltpu.CompilerParams(use_tc_tiling_on_sc=False,
                                             needs_layout_passes=False),
    )
    return f(state2, tb)[:, _UNPERM]


# speedup vs baseline: 1.9374x; 1.9374x over previous
import jax
import jax.numpy as jnp
from jax import lax
from jax.experimental import pallas as pl
from jax.experimental.pallas import tpu as pltpu
from jax.experimental.pallas import tpu_sc as plsc

N = 16384
W = 200
D = 64
DPK = 32
L = 16
NC, NS = 2, 16
NW = NC * NS
ROWS_PER_W = N // NW
CHUNK = 32
NCHUNK = ROWS_PER_W // CHUNK
NVD = D // L
NSLOT = 8


def _fill_idx(idx_buf, j, idxv):
    for v in range(W // L):
        sl = pl.ds(v * L, L)
        idxv[sl] = idx_buf[j, sl] + 1
    tl = pl.ds(W - L, L)
    idxv[tl] = idx_buf[j, tl] + 1


def _gather_start(table_hbm, idxv, rows_ref, sem):
    pltpu.make_async_copy(table_hbm.at[idxv], rows_ref, sem).start()


def _gather_wait(table_hbm, idxv, rows_ref, sem):
    pltpu.make_async_copy(table_hbm.at[idxv], rows_ref, sem).wait()


def _reduce_row(rows_ref, out_ref, r):
    def body(w, accs):
        a = list(accs)
        for d in range(NVD):
            a[d] = a[d] + rows_ref[2 * w, pl.ds(d * L, L)]
        for d in range(NVD):
            a[NVD + d] = a[NVD + d] + rows_ref[2 * w + 1, pl.ds(d * L, L)]
        return tuple(a)

    z = jnp.zeros((L,), jnp.float32)
    accs = lax.fori_loop(0, W // 2, body, (z,) * (2 * NVD), unroll=4)
    scale = jnp.float32(1.0 / W)
    for d in range(NVD):
        out_ref[r, pl.ds(d * L, L)] = (accs[d] + accs[NVD + d]) * scale


def _sc_body(state_hbm, table_hbm, out_hbm, idx_buf, out_buf, *rest):
    rows = rest[:NSLOT]
    idxvs = rest[NSLOT:2 * NSLOT]
    sems = rest[2 * NSLOT:]
    wid = lax.axis_index("s") * NC + lax.axis_index("c")
    base = wid * ROWS_PER_W

    def chunk_body(c, _):
        row0 = base + c * CHUNK
        pltpu.sync_copy(state_hbm.at[pl.ds(row0, CHUNK)], idx_buf)

        for k in range(NSLOT):
            _fill_idx(idx_buf, k, idxvs[k])
            _gather_start(table_hbm, idxvs[k], rows[k], sems[k])

        def ring(i, _):
            for k in range(NSLOT):
                r = NSLOT * i + k
                _gather_wait(table_hbm, idxvs[k], rows[k], sems[k])
                @pl.when(i < CHUNK // NSLOT - 1)
                def _():
                    _fill_idx(idx_buf, r + NSLOT, idxvs[k])
                    _gather_start(table_hbm, idxvs[k], rows[k], sems[k])
                _reduce_row(rows[k], out_buf, r)
            return 0
        lax.fori_loop(0, CHUNK // NSLOT, ring, 0)

        pltpu.sync_copy(out_buf, out_hbm.at[pl.ds(row0, CHUNK)])
        return 0

    lax.fori_loop(0, NCHUNK, chunk_body, 0)


def kernel(state, table):
    state2 = state.reshape(N, 2 * W).astype(jnp.int32)
    f = pl.kernel(
        _sc_body,
        out_type=jax.ShapeDtypeStruct((N, D), jnp.float32),
        mesh=plsc.VectorSubcoreMesh(core_axis_name="c", subcore_axis_name="s"),
        scratch_types=[
            pltpu.VMEM((CHUNK, 2 * W), jnp.int32),
            pltpu.VMEM((CHUNK, D), jnp.float32),
        ] + [pltpu.VMEM((W, D), jnp.float32)] * NSLOT
          + [pltpu.VMEM((W,), jnp.int32)] * NSLOT
          + [pltpu.SemaphoreType.DMA] * NSLOT,
        compiler_params=pltpu.CompilerParams(use_tc_tiling_on_sc=False),
    )
    return f(state2, table)


# speedup vs baseline: 2.0740x; 1.0705x over previous
import jax
import jax.numpy as jnp
from jax import lax
from jax.experimental import pallas as pl
from jax.experimental.pallas import tpu as pltpu
from jax.experimental.pallas import tpu_sc as plsc

N = 16384
W = 200
D = 64
L = 16
NC, NS = 2, 16
NW = NC * NS
ROWS_PER_W = N // NW
CHUNK = 64
NCHUNK = ROWS_PER_W // CHUNK
WPAD = 208
G1 = 128
G2 = W - G1
NVD = D // L
NSLOT = 4


def _gather_start(table_hbm, idx_ref, j, rows_ref, sem_a, sem_b):
    pltpu.make_async_copy(
        table_hbm.at[idx_ref.at[j, pl.ds(0, G1)]],
        rows_ref.at[pl.ds(0, G1)], sem_a).start()
    pltpu.make_async_copy(
        table_hbm.at[idx_ref.at[j, pl.ds(G1, G2)]],
        rows_ref.at[pl.ds(G1, G2)], sem_b).start()


def _gather_wait(table_hbm, idx_ref, j, rows_ref, sem_a, sem_b):
    pltpu.make_async_copy(
        table_hbm.at[idx_ref.at[j, pl.ds(0, G1)]],
        rows_ref.at[pl.ds(0, G1)], sem_a).wait()
    pltpu.make_async_copy(
        table_hbm.at[idx_ref.at[j, pl.ds(G1, G2)]],
        rows_ref.at[pl.ds(G1, G2)], sem_b).wait()


def _reduce_row(rows_ref, out_ref, r):
    def body(w, accs):
        a = list(accs)
        for d in range(NVD):
            a[d] = a[d] + rows_ref[2 * w, pl.ds(d * L, L)]
        for d in range(NVD):
            a[NVD + d] = a[NVD + d] + rows_ref[2 * w + 1, pl.ds(d * L, L)]
        return tuple(a)

    z = jnp.zeros((L,), jnp.float32)
    accs = lax.fori_loop(0, W // 2, body, (z,) * (2 * NVD), unroll=4)
    scale = jnp.float32(1.0 / W)
    for d in range(NVD):
        out_ref[r, pl.ds(d * L, L)] = (accs[d] + accs[NVD + d]) * scale


def _sc_body(state_hbm, table_hbm, out_hbm, idx_buf, out_buf, *rest):
    rows = rest[:NSLOT]
    flat_sems = rest[NSLOT:]
    sems = tuple((flat_sems[2 * k], flat_sems[2 * k + 1])
                 for k in range(NSLOT))
    wid = lax.axis_index("s") * NC + lax.axis_index("c")
    base = wid * ROWS_PER_W

    def chunk_body(c, _):
        row0 = base + c * CHUNK
        pltpu.sync_copy(state_hbm.at[pl.ds(row0, CHUNK), pl.ds(0, W)],
                        idx_buf.at[pl.ds(0, CHUNK), pl.ds(0, W)])

        def plus1(j, _):
            for v in range(WPAD // L):
                sl = pl.ds(v * L, L)
                idx_buf[j, sl] = idx_buf[j, sl] + 1
            return 0
        lax.fori_loop(0, CHUNK, plus1, 0)

        for k in range(NSLOT):
            _gather_start(table_hbm, idx_buf, k, rows[k], *sems[k])

        def ring(i, _):
            for k in range(NSLOT):
                r = NSLOT * i + k
                _gather_wait(table_hbm, idx_buf, r, rows[k], *sems[k])
                @pl.when(i < CHUNK // NSLOT - 1)
                def _():
                    _gather_start(table_hbm, idx_buf, r + NSLOT, rows[k],
                                  *sems[k])
                _reduce_row(rows[k], out_buf, r)
            return 0
        lax.fori_loop(0, CHUNK // NSLOT, ring, 0)

        pltpu.sync_copy(out_buf, out_hbm.at[pl.ds(row0, CHUNK)])
        return 0

    lax.fori_loop(0, NCHUNK, chunk_body, 0)


def kernel(state, table):
    state2 = state.reshape(N, 2 * W).astype(jnp.int32)
    f = pl.kernel(
        _sc_body,
        out_type=jax.ShapeDtypeStruct((N, D), jnp.float32),
        mesh=plsc.VectorSubcoreMesh(core_axis_name="c", subcore_axis_name="s"),
        scratch_types=[
            pltpu.VMEM((CHUNK, WPAD), jnp.int32),
            pltpu.VMEM((CHUNK, D), jnp.float32),
        ] + [pltpu.VMEM((W, D), jnp.float32)] * NSLOT
          + [pltpu.SemaphoreType.DMA] * (2 * NSLOT),
        compiler_params=pltpu.CompilerParams(use_tc_tiling_on_sc=False),
    )
    return f(state2, table)
